# baseline (speedup 1.0000x reference)
import functools
import math

import jax
import jax.numpy as jnp
from jax import lax
from jax.experimental import pallas as pl
from jax.experimental.pallas import tpu as pltpu
from jax.experimental.pallas import tpu_sc as plsc

_NUM_EMBEDDINGS = 1000000
_DIM = 32
_BATCH = 16384
_HIST = 50
_SCALE = math.sqrt(float(_DIM))

_NC = 2
_NS = 16
_NW = _NC * _NS

_B = _BATCH * _HIST
_B_PER_W = _B // _NW
_GROUP = 1024
_NG = _B_PER_W // _GROUP

_T1_ROWS = _NUM_EMBEDDINGS * _DIM // 128
_T1_BLK = 2000
_T1_GRID = _T1_ROWS // _T1_BLK


def _t1_body(x_ref, o_ref):
    o_ref[...] = (x_ref[...] * _SCALE).astype(jnp.bfloat16)


_t1_scale_cast = pl.pallas_call(
    _t1_body,
    grid=(_T1_GRID,),
    in_specs=[pl.BlockSpec((_T1_BLK, 128), lambda i: (i, 0))],
    out_specs=pl.BlockSpec((_T1_BLK, 128), lambda i: (i, 0)),
    out_shape=jax.ShapeDtypeStruct((_T1_ROWS, 128), jnp.bfloat16),
)

_T2_ROWS = _B * _DIM // 128
_T2_BLK = 2048
_T2_GRID = _T2_ROWS // _T2_BLK


def _t2_body(x_ref, o_ref):
    o_ref[...] = x_ref[...].astype(jnp.float32)


_t2_widen = pl.pallas_call(
    _t2_body,
    grid=(_T2_GRID,),
    in_specs=[pl.BlockSpec((_T2_BLK, 128), lambda i: (i, 0))],
    out_specs=pl.BlockSpec((_T2_BLK, 128), lambda i: (i, 0)),
    out_shape=jax.ShapeDtypeStruct((_T2_ROWS, 128), jnp.float32),
)


def _make_emb():
    mesh = plsc.VectorSubcoreMesh(core_axis_name="c", subcore_axis_name="s")

    @functools.partial(
        pl.kernel,
        mesh=mesh,
        out_type=jax.ShapeDtypeStruct((_B // _GROUP, _GROUP, _DIM), jnp.bfloat16),
        compiler_params=pltpu.CompilerParams(use_tc_tiling_on_sc=False),
        scratch_types=[
            pltpu.VMEM((_NG, _GROUP), jnp.int32),
            pltpu.VMEM((2, _GROUP, _DIM), jnp.bfloat16),
            pltpu.SemaphoreType.DMA((2,)),
            pltpu.SemaphoreType.DMA((2,)),
        ],
    )
    def emb(idx_hbm, table_hbm, out_hbm, idx_v, gbuf, gsem, osem):
        wid = lax.axis_index("s") * _NC + lax.axis_index("c")
        gbase = wid * _NG
        pltpu.sync_copy(idx_hbm.at[wid], idx_v)

        def gather_desc(g, h):
            return pltpu.make_async_copy(
                table_hbm.at[idx_v.at[g]],
                gbuf.at[h],
                gsem.at[h],
            )

        def store_desc(g, h):
            return pltpu.make_async_copy(
                gbuf.at[h],
                out_hbm.at[gbase + g],
                osem.at[h],
            )

        gather_desc(0, 0).start()

        def group_body(g, carry):
            h = lax.rem(g, 2)
            hn = lax.rem(g + 1, 2)

            @pl.when(g >= 1)
            def _():
                store_desc(g - 1, hn).wait()

            @pl.when(g + 1 < _NG)
            def _():
                gather_desc(g + 1, hn).start()

            gather_desc(g, h).wait()
            store_desc(g, h).start()
            return carry

        lax.fori_loop(0, _NG, group_body, 0)
        store_desc(_NG - 1, (_NG - 1) % 2).wait()

    return emb


_emb = _make_emb()


@jax.jit
def kernel(inputs, table):
    scaled = _t1_scale_cast(table.reshape(_T1_ROWS, 128))
    scaled = scaled.reshape(_NUM_EMBEDDINGS, _DIM)
    idx = inputs.reshape(_NW, _NG, _GROUP)
    out16 = _emb(idx, scaled)
    out = _t2_widen(out16.reshape(_T2_ROWS, 128))
    return out.reshape(_BATCH, _HIST, _DIM)


# speedup vs baseline: 1.5810x; 1.5810x over previous
import functools
import math

import jax
import jax.numpy as jnp
from jax import lax
from jax.experimental import pallas as pl
from jax.experimental.pallas import tpu as pltpu
from jax.experimental.pallas import tpu_sc as plsc

_NUM_EMBEDDINGS = 1000000
_DIM = 32
_BATCH = 16384
_HIST = 50
_SCALE = math.sqrt(float(_DIM))

_L = 16
_NC = 2
_NS = 16
_NW = _NC * _NS

_ROWS_PER_W = _BATCH // _NW
_GR = 16
_GROUP = _GR * _HIST
_NG = _ROWS_PER_W // _GR


def _make_emb():
    mesh = plsc.VectorSubcoreMesh(core_axis_name="c", subcore_axis_name="s")

    @functools.partial(
        pl.kernel,
        mesh=mesh,
        out_type=jax.ShapeDtypeStruct((_BATCH, _HIST, _DIM), jnp.float32),
        compiler_params=pltpu.CompilerParams(use_tc_tiling_on_sc=False),
        scratch_types=[
            pltpu.VMEM((_ROWS_PER_W, _HIST), jnp.int32),
            pltpu.VMEM((2, _GROUP, _DIM), jnp.float32),
            pltpu.SemaphoreType.DMA((2,)),
            pltpu.SemaphoreType.DMA((2,)),
        ],
    )
    def emb(idx_hbm, table_hbm, out_hbm, idx_v, gbuf, gsem, osem):
        wid = lax.axis_index("s") * _NC + lax.axis_index("c")
        rbase = wid * _ROWS_PER_W
        pltpu.sync_copy(idx_hbm.at[pl.ds(rbase, _ROWS_PER_W)], idx_v)

        def gather_descs(g, h):
            return [
                pltpu.make_async_copy(
                    table_hbm.at[idx_v.at[g * _GR + k]],
                    gbuf.at[h].at[pl.ds(k * _HIST, _HIST)],
                    gsem.at[h],
                )
                for k in range(_GR)
            ]

        def store_descs(g, h):
            return [
                pltpu.make_async_copy(
                    gbuf.at[h].at[pl.ds(k * _HIST, _HIST)],
                    out_hbm.at[rbase + g * _GR + k],
                    osem.at[h],
                )
                for k in range(_GR)
            ]

        for d in gather_descs(0, 0):
            d.start()

        def group_body(g, carry):
            h = lax.rem(g, 2)
            hn = lax.rem(g + 1, 2)

            @pl.when(g >= 1)
            def _():
                for d in store_descs(g - 1, hn):
                    d.wait()

            @pl.when(g + 1 < _NG)
            def _():
                for d in gather_descs(g + 1, hn):
                    d.start()

            for d in gather_descs(g, h):
                d.wait()

            @plsc.parallel_loop(0, _GROUP, step=1, unroll=8)
            def _scale(r):
                gbuf[h, r, pl.ds(0, _L)] = gbuf[h, r, pl.ds(0, _L)] * _SCALE
                gbuf[h, r, pl.ds(_L, _L)] = gbuf[h, r, pl.ds(_L, _L)] * _SCALE

            for d in store_descs(g, h):
                d.start()
            return carry

        lax.fori_loop(0, _NG, group_body, 0)
        for d in store_descs(_NG - 1, (_NG - 1) % 2):
            d.wait()

    return emb


_emb = _make_emb()


@jax.jit
def kernel(inputs, table):
    return _emb(inputs, table)
